# baseline (speedup 1.0000x reference)
import jax
import jax.numpy as jnp
from jax.experimental import pallas as pl
from jax.experimental.pallas import tpu as pltpu

MAX_LEN = 4096
D_MODEL = 1024
BATCH = 4

CHUNKS = (1024, 1024, 1024, 1024)
OFFSETS = tuple(sum(CHUNKS[:i]) for i in range(len(CHUNKS)))
NUM_CHUNKS = len(CHUNKS)


def _dma_body(table_hbm, out_hbm, buf, rsems, wsems):
    def read(c):
        h = pltpu.make_async_copy(
            table_hbm.at[pl.ds(OFFSETS[c], CHUNKS[c]), :],
            buf.at[pl.ds(OFFSETS[c], CHUNKS[c]), :], rsems.at[c])
        h.start()
        return h

    def write(c, b):
        h = pltpu.make_async_copy(
            buf.at[pl.ds(OFFSETS[c], CHUNKS[c]), :],
            out_hbm.at[b, pl.ds(OFFSETS[c], CHUNKS[c]), :],
            wsems.at[c])
        h.start()
        return h

    reads = [read(c) for c in range(NUM_CHUNKS)]
    writes = []
    for c in range(NUM_CHUNKS):
        reads[c].wait()
        writes += [write(c, (c + j) % BATCH) for j in range(BATCH)]
    for h in writes:
        h.wait()


@jax.jit
def _broadcast_table(emb_weight):
    return pl.pallas_call(
        _dma_body,
        in_specs=[pl.BlockSpec(memory_space=pltpu.MemorySpace.HBM)],
        out_specs=pl.BlockSpec(memory_space=pltpu.MemorySpace.HBM),
        out_shape=jax.ShapeDtypeStruct((BATCH, MAX_LEN, D_MODEL), jnp.float32),
        scratch_shapes=[
            pltpu.VMEM((MAX_LEN, D_MODEL), jnp.float32),
            pltpu.SemaphoreType.DMA((NUM_CHUNKS,)),
            pltpu.SemaphoreType.DMA((NUM_CHUNKS,)),
        ],
    )(emb_weight)


def kernel(x, emb_weight):
    del x
    return _broadcast_table(emb_weight)


# speedup vs baseline: 1.0024x; 1.0024x over previous
import jax
import jax.numpy as jnp
from jax.experimental import pallas as pl
from jax.experimental.pallas import tpu as pltpu

MAX_LEN = 4096
D_MODEL = 1024
BATCH = 4

CHUNKS = (1024, 1024, 1024, 1024)
OFFSETS = tuple(sum(CHUNKS[:i]) for i in range(len(CHUNKS)))
NUM_CHUNKS = len(CHUNKS)


def _dma_body(table_hbm, out_hbm, buf, rsems, wsems):
    def read(c):
        h = pltpu.make_async_copy(
            table_hbm.at[pl.ds(OFFSETS[c], CHUNKS[c]), :],
            buf.at[pl.ds(OFFSETS[c], CHUNKS[c]), :], rsems.at[c])
        h.start()
        return h

    def write(c, b):
        h = pltpu.make_async_copy(
            buf.at[pl.ds(OFFSETS[c], CHUNKS[c]), :],
            out_hbm.at[b, pl.ds(OFFSETS[c], CHUNKS[c]), :],
            wsems.at[c])
        h.start()
        return h

    reads = [read(c) for c in range(NUM_CHUNKS)]
    writes = []
    for c in range(NUM_CHUNKS):
        reads[c].wait()
        writes += [write(c, b) for b in range(BATCH)]
    for h in writes:
        h.wait()


@jax.jit
def _broadcast_table(emb_weight):
    return pl.pallas_call(
        _dma_body,
        in_specs=[pl.BlockSpec(memory_space=pltpu.MemorySpace.HBM)],
        out_specs=pl.BlockSpec(memory_space=pltpu.MemorySpace.HBM),
        out_shape=jax.ShapeDtypeStruct((BATCH, MAX_LEN, D_MODEL), jnp.float32),
        scratch_shapes=[
            pltpu.VMEM((MAX_LEN, D_MODEL), jnp.float32),
            pltpu.SemaphoreType.DMA((NUM_CHUNKS,)),
            pltpu.SemaphoreType.DMA((NUM_CHUNKS,)),
        ],
    )(emb_weight)


def kernel(x, emb_weight):
    del x
    return _broadcast_table(emb_weight)
